# speedup vs baseline: 2.1105x; 2.1105x over previous
import functools

import jax
import jax.numpy as jnp
from jax import lax
from jax.experimental import pallas as pl
from jax.experimental.pallas import tpu as pltpu

D = 768
P = 8
E = 4096
RB = 256


def _fused_body(idx_ref, zs_ref, zd_ref, wp_ref, bp_ref, wb_ref, bb_ref, out_ref):
    zs = zs_ref[...]
    zd = zd_ref[...]
    idx = idx_ref[...]
    u = lax.dot_general(zd, wb_ref[0], (((1,), (1,)), ((), ())),
                        preferred_element_type=jnp.float32)
    acc = jnp.zeros((RB, 1), jnp.float32)
    for p in range(P):
        prj = lax.dot_general(zs, wp_ref[p], (((1,), (1,)), ((), ())),
                              preferred_element_type=jnp.float32)
        prj = prj + bp_ref[p][None, :]
        s_p = jnp.sum(prj * u, axis=1, keepdims=True)
        acc = jnp.where(idx == p, s_p, acc)
    out_ref[...] = acc + bb_ref[0, 0]


def kernel(z_src, z_dst, lr_pair_idx, W_proj, b_proj, W_bil, b_bil):
    idx = lr_pair_idx.astype(jnp.int32).reshape(E, 1)
    bb = b_bil.astype(jnp.float32).reshape(1, 1)
    grid = (E // RB,)
    out = pl.pallas_call(
        _fused_body,
        grid=grid,
        in_specs=[
            pl.BlockSpec((RB, 1), lambda b: (b, 0)),
            pl.BlockSpec((RB, D), lambda b: (b, 0)),
            pl.BlockSpec((RB, D), lambda b: (b, 0)),
            pl.BlockSpec((P, D, D), lambda b: (0, 0, 0)),
            pl.BlockSpec((P, D), lambda b: (0, 0)),
            pl.BlockSpec((1, D, D), lambda b: (0, 0, 0)),
            pl.BlockSpec(memory_space=pltpu.SMEM),
        ],
        out_specs=pl.BlockSpec((RB, 1), lambda b: (b, 0)),
        out_shape=jax.ShapeDtypeStruct((E, 1), jnp.float32),
    )(idx, z_src, z_dst, W_proj, b_proj, W_bil, bb)
    return out
